# Initial kernel scaffold; baseline (speedup 1.0000x reference)
import jax
import jax.numpy as jnp
from jax.experimental import pallas as pl


def kernel(coords, x_emb, y_emb, z_emb, W, b):
    raise NotImplementedError("write your pallas kernel here")


# speedup vs baseline: 2.7552x; 2.7552x over previous
import jax
import jax.numpy as jnp
from jax.experimental import pallas as pl
from jax.experimental.pallas import tpu as pltpu

HID = 96
PER = 32
NROWS = 30
BLK = 4096


def _tc_body(coords_ref, xe_ref, ye_ref, ze_ref, w_ref, b_ref, out_ref, ttb_ref):
    @pl.when(pl.program_id(0) == 0)
    def _init():
        w = w_ref[...]
        dn = (((1,), (1,)), ((), ()))
        tx = jax.lax.dot_general(xe_ref[...], w[:, 0:32], dn,
                                 preferred_element_type=jnp.float32)
        ty = jax.lax.dot_general(ye_ref[...], w[:, 32:64], dn,
                                 preferred_element_type=jnp.float32)
        tz = jax.lax.dot_general(ze_ref[...], w[:, 64:96], dn,
                                 preferred_element_type=jnp.float32)
        ttb_ref[...] = jnp.zeros((128, HID), jnp.bfloat16)
        ttb_ref[0:30, :] = (tx + b_ref[...][None, :]).astype(jnp.bfloat16)
        ttb_ref[32:62, :] = ty.astype(jnp.bfloat16)
        ttb_ref[64:94, :] = tz.astype(jnp.bfloat16)

    c = coords_ref[...]

    def idx(col):
        v = jnp.clip(jnp.round(c[:, col:col + 1]), 0.0, 29.0)
        return v.astype(jnp.int32)

    ix = idx(0)
    iy = idx(1) + 32
    iz = idx(2) + 64
    col = jax.lax.broadcasted_iota(jnp.int32, (BLK, 128), 1)
    sel = jnp.where(col < 32, ix, jnp.where(col < 64, iy, iz))
    oh = (col == sel).astype(jnp.bfloat16)
    out_ref[...] = jax.lax.dot_general(
        oh, ttb_ref[...], (((1,), (0,)), ((), ())),
        preferred_element_type=jnp.float32)


def kernel(coords, x_emb, y_emb, z_emb, W, b):
    B, S, _ = coords.shape
    n = B * S
    cf = coords.reshape(n, 3)
    out = pl.pallas_call(
        _tc_body,
        grid=(n // BLK,),
        in_specs=[
            pl.BlockSpec((BLK, 3), lambda i: (i, 0)),
            pl.BlockSpec((NROWS, PER), lambda i: (0, 0)),
            pl.BlockSpec((NROWS, PER), lambda i: (0, 0)),
            pl.BlockSpec((NROWS, PER), lambda i: (0, 0)),
            pl.BlockSpec((HID, HID), lambda i: (0, 0)),
            pl.BlockSpec((HID,), lambda i: (0,)),
        ],
        out_specs=pl.BlockSpec((BLK, HID), lambda i: (i, 0)),
        out_shape=jax.ShapeDtypeStruct((n, HID), jnp.float32),
        scratch_shapes=[pltpu.VMEM((128, HID), jnp.bfloat16)],
    )(cf, x_emb, y_emb, z_emb, W, b)
    return out.reshape(B, S, HID)


# speedup vs baseline: 15.1806x; 5.5099x over previous
import jax
import jax.numpy as jnp
from jax.experimental import pallas as pl
from jax.experimental.pallas import tpu as pltpu

HID = 96
PER = 32
NROWS = 30
BLK = 4096


def _tc_body(cx_ref, cy_ref, cz_ref, xe_ref, ye_ref, ze_ref, w_ref, b_ref,
             out_ref, ttb_ref):
    @pl.when(pl.program_id(0) == 0)
    def _init():
        w = w_ref[...]
        dn = (((1,), (1,)), ((), ()))
        tx = jax.lax.dot_general(xe_ref[...], w[:, 0:32], dn,
                                 preferred_element_type=jnp.float32)
        ty = jax.lax.dot_general(ye_ref[...], w[:, 32:64], dn,
                                 preferred_element_type=jnp.float32)
        tz = jax.lax.dot_general(ze_ref[...], w[:, 64:96], dn,
                                 preferred_element_type=jnp.float32)
        ttb_ref[...] = jnp.zeros((128, HID), jnp.bfloat16)
        ttb_ref[0:30, :] = (tx + b_ref[...][None, :]).astype(jnp.bfloat16)
        ttb_ref[32:62, :] = ty.astype(jnp.bfloat16)
        ttb_ref[64:94, :] = tz.astype(jnp.bfloat16)

    def toidx(ref):
        return jnp.clip(jnp.round(ref[...]), 0.0, 29.0).astype(jnp.int32)

    ix = toidx(cx_ref)
    iy = toidx(cy_ref) + 32
    iz = toidx(cz_ref) + 64
    rowc = jax.lax.broadcasted_iota(jnp.int32, (128, 128), 0)
    chunks = []
    for j in range(SUBB):
        m = ((rowc == ix[j:j + 1, :]) | (rowc == iy[j:j + 1, :])
             | (rowc == iz[j:j + 1, :]))
        chunks.append(m)
    ohT = jnp.concatenate(chunks, axis=1).astype(jnp.bfloat16)
    out_ref[...] = jax.lax.dot_general(
        ohT, ttb_ref[...], (((0,), (0,)), ((), ())),
        preferred_element_type=jnp.float32)


SUBB = BLK // 128


def kernel(coords, x_emb, y_emb, z_emb, W, b):
    B, S, _ = coords.shape
    n = B * S
    nr = n // 128
    cx = coords[..., 0].reshape(nr, 128)
    cy = coords[..., 1].reshape(nr, 128)
    cz = coords[..., 2].reshape(nr, 128)
    cspec = pl.BlockSpec((SUBB, 128), lambda i: (i, 0))
    out = pl.pallas_call(
        _tc_body,
        grid=(n // BLK,),
        in_specs=[
            cspec, cspec, cspec,
            pl.BlockSpec((NROWS, PER), lambda i: (0, 0)),
            pl.BlockSpec((NROWS, PER), lambda i: (0, 0)),
            pl.BlockSpec((NROWS, PER), lambda i: (0, 0)),
            pl.BlockSpec((HID, HID), lambda i: (0, 0)),
            pl.BlockSpec((HID,), lambda i: (0,)),
        ],
        out_specs=pl.BlockSpec((BLK, HID), lambda i: (i, 0)),
        out_shape=jax.ShapeDtypeStruct((n, HID), jnp.float32),
        scratch_shapes=[pltpu.VMEM((128, HID), jnp.bfloat16)],
    )(cx, cy, cz, x_emb, y_emb, z_emb, W, b)
    return out.reshape(B, S, HID)
